# baseline (speedup 1.0000x reference)
import jax
import jax.numpy as jnp
from jax.experimental import pallas as pl
from jax.experimental.pallas import tpu as pltpu

D_MODEL = 4096
NUM_EXPERTS = 64
TOP_K = 8
BLOCK_T = 1024


def _gate_body(x_ref, wt_ref, bt_ref, vals_ref, idx_ref):
    logits_t = jax.lax.dot_general(
        wt_ref[...], x_ref[...],
        dimension_numbers=(((1,), (1,)), ((), ())),
        preferred_element_type=jnp.float32,
    ) + bt_ref[...]
    m = jnp.max(logits_t, axis=0, keepdims=True)
    e = jnp.exp(logits_t - m)
    probs = e / jnp.sum(e, axis=0, keepdims=True)

    bits = jax.lax.bitcast_convert_type(probs, jnp.int32)
    iota = jax.lax.broadcasted_iota(jnp.int32, probs.shape, 0)
    key = (bits & ~0x3F) | (NUM_EXPERTS - 1 - iota)
    keys = []
    work = key
    for _ in range(TOP_K):
        mx = jnp.max(work, axis=0, keepdims=True)
        keys.append(mx)
        work = jnp.where(work == mx, -1, work)
    top = jnp.concatenate(keys, axis=0)
    top_t = jnp.transpose(top, (1, 0))
    idx_ref[...] = (NUM_EXPERTS - 1) - (top_t & 0x3F)
    vals_ref[...] = jax.lax.bitcast_convert_type(top_t & ~0x3F, jnp.float32)


@jax.jit
def kernel(x, W_gate, b_gate):
    n_tokens = x.shape[0]
    grid = (n_tokens // BLOCK_T,)
    wt = W_gate.T
    bt = b_gate.reshape(NUM_EXPERTS, 1)
    out_vals, out_idx = pl.pallas_call(
        _gate_body,
        grid=grid,
        in_specs=[
            pl.BlockSpec((BLOCK_T, D_MODEL), lambda i: (i, 0)),
            pl.BlockSpec((NUM_EXPERTS, D_MODEL), lambda i: (0, 0)),
            pl.BlockSpec((NUM_EXPERTS, 1), lambda i: (0, 0)),
        ],
        out_specs=[
            pl.BlockSpec((BLOCK_T, TOP_K), lambda i: (i, 0)),
            pl.BlockSpec((BLOCK_T, TOP_K), lambda i: (i, 0)),
        ],
        out_shape=[
            jax.ShapeDtypeStruct((n_tokens, TOP_K), jnp.float32),
            jax.ShapeDtypeStruct((n_tokens, TOP_K), jnp.int32),
        ],
        compiler_params=pltpu.CompilerParams(
            dimension_semantics=("parallel",),
        ),
    )(x, wt, bt)
    return out_vals, out_idx


# speedup vs baseline: 1.1878x; 1.1878x over previous
import jax
import jax.numpy as jnp
from jax.experimental import pallas as pl
from jax.experimental.pallas import tpu as pltpu

D_MODEL = 4096
NUM_EXPERTS = 64
TOP_K = 8
BLOCK_T = 1024
HALF_D = D_MODEL // 2


def _gate_body_s(xa_ref, xb_ref, wt_ref, bt_ref, vals_ref, idx_ref):
    dn = (((1,), (1,)), ((), ()))
    logits_t = jax.lax.dot_general(
        wt_ref[:, :HALF_D], xa_ref[...], dimension_numbers=dn,
        preferred_element_type=jnp.float32,
    )
    logits_t = logits_t + jax.lax.dot_general(
        wt_ref[:, HALF_D:], xb_ref[...], dimension_numbers=dn,
        preferred_element_type=jnp.float32,
    )
    logits_t = logits_t + bt_ref[...]
    m = jnp.max(logits_t, axis=0, keepdims=True)
    e = jnp.exp(logits_t - m)
    denom = jnp.sum(e, axis=0, keepdims=True)

    bits = jax.lax.bitcast_convert_type(e, jnp.int32)
    iota = jax.lax.broadcasted_iota(jnp.int32, e.shape, 0)
    key = (bits & ~0x3F) | (NUM_EXPERTS - 1 - iota)
    keys = []
    work = key
    for _ in range(TOP_K):
        mx = jnp.max(work, axis=0, keepdims=True)
        keys.append(mx)
        work = jnp.where(work == mx, -1, work)
    top = jnp.concatenate(keys, axis=0)
    idx_ref[...] = (NUM_EXPERTS - 1) - (top & 0x3F)
    vals_ref[...] = (
        jax.lax.bitcast_convert_type(top & ~0x3F, jnp.float32) / denom
    )


@jax.jit
def kernel(x, W_gate, b_gate):
    n_tokens = x.shape[0]
    grid = (n_tokens // BLOCK_T,)
    wt = W_gate.T
    bt = b_gate.reshape(NUM_EXPERTS, 1)
    vals_t, idx_t = pl.pallas_call(
        _gate_body_s,
        grid=grid,
        in_specs=[
            pl.BlockSpec((BLOCK_T, HALF_D), lambda i: (i, 0)),
            pl.BlockSpec((BLOCK_T, HALF_D), lambda i: (i, 1)),
            pl.BlockSpec((NUM_EXPERTS, D_MODEL), lambda i: (0, 0)),
            pl.BlockSpec((NUM_EXPERTS, 1), lambda i: (0, 0)),
        ],
        out_specs=[
            pl.BlockSpec((TOP_K, BLOCK_T), lambda i: (0, i)),
            pl.BlockSpec((TOP_K, BLOCK_T), lambda i: (0, i)),
        ],
        out_shape=[
            jax.ShapeDtypeStruct((TOP_K, n_tokens), jnp.float32),
            jax.ShapeDtypeStruct((TOP_K, n_tokens), jnp.int32),
        ],
        compiler_params=pltpu.CompilerParams(
            dimension_semantics=("parallel",),
        ),
    )(x, x, wt, bt)
    return vals_t.T, idx_t.T
